# Initial kernel scaffold; baseline (speedup 1.0000x reference)
import jax
import jax.numpy as jnp
from jax.experimental import pallas as pl


def kernel(x, edge_index1, edge_index2, t1, W1a, b1a, g1a, be1a, W1b, b1b, ng, nb, t2, W2a, b2a, g2a, be2a, W2b, b2b):
    raise NotImplementedError("write your pallas kernel here")


# speedup vs baseline: 7.5851x; 7.5851x over previous
import functools

import jax
import jax.numpy as jnp
from jax.experimental import pallas as pl
from jax.experimental.pallas import tpu as pltpu

N0, N1, N2 = 10000, 5000, 2500
D, HID, OUT = 128, 256, 64
K1 = 5120
M1 = 2560
K2 = 2560
M2 = 2560
BR = 256


def _ln(h, g, b):
    mu = jnp.mean(h, axis=-1, keepdims=True)
    var = jnp.mean((h - mu) * (h - mu), axis=-1, keepdims=True)
    return (h - mu) * jax.lax.rsqrt(var + 1e-5) * g + b


def _prep_body(x_ref, t_ref, u_ref, v_ref):
    x = x_ref[...]
    r = jnp.maximum(x, 0.0) + 1e-7
    z = t_ref[0, 0] * r
    c = jnp.max(z, axis=0, keepdims=True)
    v = jnp.exp(z - c)
    u_ref[...] = v * r
    v_ref[...] = v


def _prep(x_pad, t):
    n = x_pad.shape[0]
    return pl.pallas_call(
        _prep_body,
        out_shape=(
            jax.ShapeDtypeStruct((n, D), jnp.float32),
            jax.ShapeDtypeStruct((n, D), jnp.float32),
        ),
    )(x_pad, t.reshape(1, 1))


def _layer_body(a_ref, p_ref, xd_ref, w1_ref, b1_ref, g1_ref, be1_ref,
                w2_ref, b2_ref, ng_ref, nb_ref, o_ref, *, post):
    a = a_ref[...]
    uv = jnp.dot(a, p_ref[...], preferred_element_type=jnp.float32)
    agg = uv[:, :D] / (uv[:, D:] + 1e-16)
    h0 = agg + xd_ref[...]
    h = jnp.dot(h0, w1_ref[...], preferred_element_type=jnp.float32) + b1_ref[...]
    h = jnp.maximum(_ln(h, g1_ref[...], be1_ref[...]), 0.0)
    y = jnp.dot(h, w2_ref[...], preferred_element_type=jnp.float32) + b2_ref[...]
    if post == "gelu_ln":
        o_ref[...] = jax.nn.gelu(_ln(y, ng_ref[...], nb_ref[...]))
    else:
        m = jnp.max(y, axis=-1, keepdims=True)
        e = jnp.exp(y - m)
        o_ref[...] = y - m - jnp.log(jnp.sum(e, axis=-1, keepdims=True))


def _layer(A, P, xd, W1, b1, g1, be1, W2, b2, ng, nb, post, dout):
    m, k = A.shape
    body = functools.partial(_layer_body, post=post)
    grid = (m // BR,)
    return pl.pallas_call(
        body,
        grid=grid,
        in_specs=[
            pl.BlockSpec((BR, k), lambda i: (i, 0)),
            pl.BlockSpec((k, 2 * D), lambda i: (0, 0)),
            pl.BlockSpec((BR, D), lambda i: (i, 0)),
            pl.BlockSpec((D, HID), lambda i: (0, 0)),
            pl.BlockSpec((1, HID), lambda i: (0, 0)),
            pl.BlockSpec((1, HID), lambda i: (0, 0)),
            pl.BlockSpec((1, HID), lambda i: (0, 0)),
            pl.BlockSpec((HID, dout), lambda i: (0, 0)),
            pl.BlockSpec((1, dout), lambda i: (0, 0)),
            pl.BlockSpec((1, dout), lambda i: (0, 0)),
            pl.BlockSpec((1, dout), lambda i: (0, 0)),
        ],
        out_specs=pl.BlockSpec((BR, dout), lambda i: (i, 0)),
        out_shape=jax.ShapeDtypeStruct((m, dout), jnp.float32),
    )(A, P, xd, W1, b1.reshape(1, -1), g1.reshape(1, -1), be1.reshape(1, -1),
      W2, b2.reshape(1, -1), ng.reshape(1, -1), nb.reshape(1, -1))


def _build_counts(edge_index, m_dst, k_src):
    src = edge_index[0]
    dst = edge_index[1]
    return jnp.zeros((m_dst, k_src), jnp.float32).at[dst, src].add(
        1.0, mode="drop")


def kernel(x, edge_index1, edge_index2, t1, W1a, b1a, g1a, be1a, W1b, b1b,
           ng, nb, t2, W2a, b2a, g2a, be2a, W2b, b2b):
    x_src = jnp.concatenate(
        [x[:N1], jnp.zeros((K1 - N1, D), jnp.float32)], axis=0)
    u1, v1 = _prep(x_src, t1)
    P1 = jnp.concatenate([u1, v1], axis=1)
    A1 = _build_counts(edge_index1, M1, K1)
    hg = _layer(A1, P1, x[:M1], W1a, b1a, g1a, be1a, W1b, b1b, ng, nb,
                "gelu_ln", D)
    u2, v2 = _prep(hg, t2)
    P2 = jnp.concatenate([u2, v2], axis=1)
    A2 = _build_counts(edge_index2, M2, K2)
    out = _layer(A2, P2, hg, W2a, b2a, g2a, be2a, W2b, b2b,
                 jnp.zeros((OUT,), jnp.float32), jnp.zeros((OUT,), jnp.float32),
                 "log_softmax", OUT)
    return out[:N2]
